# baseline (speedup 1.0000x reference)
import functools

import jax
import jax.numpy as jnp
from jax import lax
from jax.experimental import pallas as pl
from jax.experimental.pallas import tpu as pltpu
from jax.experimental.pallas import tpu_sc as plsc

VOCAB = 49408
HIDDEN = 512
MAX_POS = 16
B = 4096
T = 16

N = B * T
NW = 32
ROWS_PER_W = N // NW
CHUNK = 32
CHUNKS_PER_W = ROWS_PER_W // CHUNK
NCHUNKS = N // CHUNK
LANES = 16
GROUPS = HIDDEN // LANES
NBUF = 4


def _body(table_hbm, idx_hbm, pos_hbm, out_hbm, idx_v, pos_v,
          b0, b1, b2, b3, si0, si1, si2, si3, so0, so1, so2, so3):
    bufs = (b0, b1, b2, b3)
    sin = (si0, si1, si2, si3)
    sout = (so0, so1, so2, so3)

    wid = lax.axis_index("s") * 2 + lax.axis_index("c")
    cbase = wid * CHUNKS_PER_W

    pltpu.sync_copy(idx_hbm.at[pl.ds(cbase, CHUNKS_PER_W)], idx_v)
    pltpu.sync_copy(pos_hbm, pos_v)

    def out_slice(c):
        return out_hbm.at[pl.ds((cbase + c) * CHUNK, CHUNK)]

    def start_gather(c, b):
        pltpu.async_copy(table_hbm.at[idx_v.at[c]], bufs[b], sin[b])

    def wait_gather(c, b):
        pltpu.make_async_copy(table_hbm.at[idx_v.at[c]], bufs[b], sin[b]).wait()

    def start_scatter(c, b):
        pltpu.async_copy(bufs[b], out_slice(c), sout[b])

    def wait_scatter(c, b):
        pltpu.make_async_copy(bufs[b], out_slice(c), sout[b]).wait()

    def add_pos(b):
        buf = bufs[b]

        def g_body(g, carry):
            col = pl.ds(g * LANES, LANES)
            for p in range(MAX_POS):
                pv = pos_v[p, col]
                for j in range(CHUNK // MAX_POS):
                    r = j * MAX_POS + p
                    buf[r, col] = buf[r, col] + pv
            return carry

        lax.fori_loop(0, GROUPS, g_body, 0)

    for b in range(NBUF - 1):
        start_gather(b, b)

    def loop_body(i, carry):
        for b in range(NBUF):
            c = NBUF * i + b
            wait_gather(c, b)
            add_pos(b)
            start_scatter(c, b)
            bn = (b + NBUF - 1) % NBUF
            cn = c + NBUF - 1
            if b == 0:
                @pl.when(i == 0)
                def _():
                    start_gather(NBUF - 1, NBUF - 1)

                @pl.when(jnp.logical_and(i >= 1, cn < CHUNKS_PER_W))
                def _():
                    wait_scatter(cn - NBUF, bn)
                    start_gather(cn, bn)
            else:
                @pl.when(cn < CHUNKS_PER_W)
                def _():
                    wait_scatter(cn - NBUF, bn)
                    start_gather(cn, bn)
        return carry

    lax.fori_loop(0, CHUNKS_PER_W // NBUF, loop_body, 0)

    for b in range(NBUF):
        wait_scatter(CHUNKS_PER_W - NBUF + b, b)


@jax.jit
def _embed(ids2d, token_table, position_table):
    mesh = plsc.VectorSubcoreMesh(core_axis_name="c", subcore_axis_name="s")
    k = functools.partial(
        pl.kernel,
        out_type=jax.ShapeDtypeStruct((N, HIDDEN), jnp.float32),
        mesh=mesh,
        scratch_types=[
            pltpu.VMEM((CHUNKS_PER_W, CHUNK), jnp.int32),
            pltpu.VMEM((MAX_POS, HIDDEN), jnp.float32),
        ] + [pltpu.VMEM((CHUNK, HIDDEN), jnp.float32)] * NBUF
          + [pltpu.SemaphoreType.DMA] * (2 * NBUF),
    )(_body)
    return k(token_table, ids2d, position_table)


def kernel(input_ids, token_table, position_table):
    ids2d = input_ids.astype(jnp.int32).reshape(NCHUNKS, CHUNK)
    out = _embed(ids2d, token_table, position_table)
    return out.reshape(B, T, HIDDEN)


# speedup vs baseline: 1.3398x; 1.3398x over previous
import functools

import jax
import jax.numpy as jnp
from jax import lax
from jax.experimental import pallas as pl
from jax.experimental.pallas import tpu as pltpu
from jax.experimental.pallas import tpu_sc as plsc

VOCAB = 49408
HIDDEN = 512
MAX_POS = 16
B = 4096
T = 16

N = B * T
NW = 32
ROWS_PER_W = N // NW
CHUNK = 64
CHUNKS_PER_W = ROWS_PER_W // CHUNK
NCHUNKS = N // CHUNK
LANES = 16
GROUPS = HIDDEN // LANES
NBUF = 3


def _body(table_hbm, idx_hbm, pos_hbm, out_hbm, idx_v, pos_v,
          b0, b1, b2, si0, si1, si2, so0, so1, so2):
    bufs = (b0, b1, b2)
    sin = (si0, si1, si2)
    sout = (so0, so1, so2)

    wid = lax.axis_index("s") * 2 + lax.axis_index("c")
    cbase = wid * CHUNKS_PER_W

    pltpu.sync_copy(idx_hbm.at[pl.ds(cbase, CHUNKS_PER_W)], idx_v)
    pltpu.sync_copy(pos_hbm, pos_v)

    def out_slice(c):
        return out_hbm.at[pl.ds((cbase + c) * CHUNK, CHUNK)]

    def start_gather(c, b):
        pltpu.async_copy(table_hbm.at[idx_v.at[c]], bufs[b], sin[b])

    def wait_gather(c, b):
        pltpu.make_async_copy(table_hbm.at[idx_v.at[c]], bufs[b], sin[b]).wait()

    def start_scatter(c, b):
        pltpu.async_copy(bufs[b], out_slice(c), sout[b])

    def wait_scatter(c, b):
        pltpu.make_async_copy(bufs[b], out_slice(c), sout[b]).wait()

    def add_pos(b):
        buf = bufs[b]

        @plsc.parallel_loop(0, GROUPS)
        def g_body(g):
            col = pl.ds(g * LANES, LANES)
            for p in range(MAX_POS):
                pv = pos_v[p, col]
                for j in range(CHUNK // MAX_POS):
                    r = j * MAX_POS + p
                    buf[r, col] = buf[r, col] + pv

    start_gather(0, 0)
    start_gather(1, 1)

    def loop_body(i, carry):
        c = NBUF * i
        wait_gather(c, 0)
        add_pos(0)
        start_scatter(c, 0)

        @pl.when(i >= 1)
        def _():
            wait_scatter(c - 1, 2)

        start_gather(c + 2, 2)
        wait_gather(c + 1, 1)
        add_pos(1)
        start_scatter(c + 1, 1)
        wait_scatter(c, 0)
        start_gather(c + 3, 0)
        wait_gather(c + 2, 2)
        add_pos(2)
        start_scatter(c + 2, 2)
        wait_scatter(c + 1, 1)
        start_gather(c + 4, 1)
        return carry

    lax.fori_loop(0, CHUNKS_PER_W // NBUF, loop_body, 0)

    clast = (CHUNKS_PER_W // NBUF) * NBUF
    wait_gather(clast, 0)
    add_pos(0)
    start_scatter(clast, 0)
    wait_gather(clast + 1, 1)
    add_pos(1)
    start_scatter(clast + 1, 1)
    wait_scatter(clast - 1, 2)
    wait_scatter(clast, 0)
    wait_scatter(clast + 1, 1)


@jax.jit
def _embed(ids2d, token_table, position_table):
    mesh = plsc.VectorSubcoreMesh(core_axis_name="c", subcore_axis_name="s")
    k = functools.partial(
        pl.kernel,
        out_type=jax.ShapeDtypeStruct((N, HIDDEN), jnp.float32),
        mesh=mesh,
        scratch_types=[
            pltpu.VMEM((CHUNKS_PER_W, CHUNK), jnp.int32),
            pltpu.VMEM((MAX_POS, HIDDEN), jnp.float32),
        ] + [pltpu.VMEM((CHUNK, HIDDEN), jnp.float32)] * NBUF
          + [pltpu.SemaphoreType.DMA] * (2 * NBUF),
    )(_body)
    return k(token_table, ids2d, position_table)


def kernel(input_ids, token_table, position_table):
    ids2d = input_ids.astype(jnp.int32).reshape(NCHUNKS, CHUNK)
    out = _embed(ids2d, token_table, position_table)
    return out.reshape(B, T, HIDDEN)
